# Initial kernel scaffold; baseline (speedup 1.0000x reference)
import jax
import jax.numpy as jnp
from jax.experimental import pallas as pl


def kernel(z, pos_edge_index, neg_edge_index):
    raise NotImplementedError("write your pallas kernel here")


# speedup vs baseline: 1.1496x; 1.1496x over previous
import functools

import jax
import jax.numpy as jnp
from jax import lax
from jax.experimental import pallas as pl
from jax.experimental.pallas import tpu as pltpu
from jax.experimental.pallas import tpu_sc as plsc

EPS = 1e-15
NC = 2
NS = 16
NW = NC * NS
LANES = 16
SUBBATCH = 40


def _sc_dots(z, srcs, dsts, *, chunk, interpret=False):
    n, d = z.shape
    (e_total,) = srcs.shape
    assert e_total % NW == 0
    e_per_w = e_total // NW
    assert e_per_w % chunk == 0 and chunk % SUBBATCH == 0 and chunk % LANES == 0
    n_chunks = e_per_w // chunk
    n_sub = chunk // SUBBATCH
    n_groups = chunk // LANES

    mesh = plsc.VectorSubcoreMesh(core_axis_name="c", subcore_axis_name="s",
                                  num_cores=NC, num_subcores=NS)

    @functools.partial(
        pl.kernel,
        out_type=jax.ShapeDtypeStruct((e_total,), jnp.float32),
        mesh=mesh,
        interpret=interpret,
        compiler_params=pltpu.CompilerParams(
            use_tc_tiling_on_sc=False, needs_layout_passes=False),
        scratch_types=[
            pltpu.VMEM((chunk,), jnp.int32),
            pltpu.VMEM((chunk,), jnp.int32),
            pltpu.VMEM((chunk, d), jnp.float32),
            pltpu.VMEM((chunk, d), jnp.float32),
            pltpu.VMEM((chunk,), jnp.float32),
            pltpu.SemaphoreType.DMA,
        ],
    )
    def k(z_hbm, src_hbm, dst_hbm, out_hbm, src_idx, dst_idx, src_rows,
          dst_rows, dots, sem):
        wid = lax.axis_index("s") * NC + lax.axis_index("c")
        wbase = wid * e_per_w
        lanes = lax.iota(jnp.int32, LANES)

        def do_chunk(g, _):
            base = wbase + g * chunk
            pltpu.sync_copy(src_hbm.at[pl.ds(base, chunk)], src_idx)
            pltpu.sync_copy(dst_hbm.at[pl.ds(base, chunk)], dst_idx)
            copies = []
            for j in range(n_sub):
                sl = pl.ds(j * SUBBATCH, SUBBATCH)
                copies.append(pltpu.async_copy(
                    z_hbm.at[src_idx.at[sl]], src_rows.at[sl], sem))
                copies.append(pltpu.async_copy(
                    z_hbm.at[dst_idx.at[sl]], dst_rows.at[sl], sem))
            for cp in copies:
                cp.wait()

            def do_group(g16, _):
                rows = g16 * LANES + lanes

                def jstep(jbase, acc, off):
                    col = jnp.full((LANES,), jbase + off, jnp.int32)
                    vs = plsc.load_gather(src_rows, [rows, col])
                    vd = plsc.load_gather(dst_rows, [rows, col])
                    return acc + vs * vd

                def jbody(jj, acc):
                    jbase = jj * 4
                    for off in range(4):
                        acc = jstep(jbase, acc, off)
                    return acc

                acc = lax.fori_loop(0, d // 4, jbody,
                                    jnp.zeros((LANES,), jnp.float32))
                dots[pl.ds(g16 * LANES, LANES)] = acc
                return 0

            lax.fori_loop(0, n_groups, do_group, 0)
            pltpu.sync_copy(dots, out_hbm.at[pl.ds(base, chunk)])
            return 0

        lax.fori_loop(0, n_chunks, do_chunk, 0)

    return k(z, srcs, dsts)


def _tc_loss(dots, e_pos, *, interpret=False):
    (e_total,) = dots.shape
    assert e_total == 2 * e_pos and e_pos % 128 == 0
    rows = e_pos // 128
    d3 = dots.reshape(2, rows, 128)
    inv = 1.0 / e_pos

    def body(d_ref, out_ref):
        p_pos = jax.nn.sigmoid(d_ref[0])
        p_neg = jax.nn.sigmoid(d_ref[1])
        q = jnp.maximum(1.0 - p_neg, 0.0)
        t = -jnp.log(p_pos + EPS) - jnp.log(q + EPS)
        out_ref[0, 0] = jnp.sum(t) * inv

    out = pl.pallas_call(
        body,
        out_specs=pl.BlockSpec(memory_space=pltpu.SMEM),
        out_shape=jax.ShapeDtypeStruct((1, 1), jnp.float32),
        interpret=interpret,
    )(d3)
    return out[0, 0]


def kernel(z, pos_edge_index, neg_edge_index, *, interpret=False):
    e_pos = pos_edge_index.shape[1]
    srcs = jnp.concatenate([pos_edge_index[0], neg_edge_index[0]])
    dsts = jnp.concatenate([pos_edge_index[1], neg_edge_index[1]])
    e_total = srcs.shape[0]
    e_per_w = e_total // NW
    chunk = 400 if e_per_w % 400 == 0 else e_per_w
    dots = _sc_dots(z, srcs, dsts, chunk=chunk, interpret=interpret)
    return _tc_loss(dots, e_pos, interpret=interpret)


# speedup vs baseline: 3.3653x; 2.9273x over previous
import functools

import jax
import jax.numpy as jnp
from jax import lax
from jax.experimental import pallas as pl
from jax.experimental.pallas import tpu as pltpu
from jax.experimental.pallas import tpu_sc as plsc

EPS = 1e-15
NC = 2
NS = 16
NW = NC * NS
LANES = 16
SUBBATCH = 40


def _sc_dots(z, srcs, dsts, *, chunk, interpret=False):
    n, d = z.shape
    (e_total,) = srcs.shape
    assert e_total % NW == 0
    e_per_w = e_total // NW
    assert e_per_w % chunk == 0 and chunk % SUBBATCH == 0 and chunk % LANES == 0
    n_chunks = e_per_w // chunk
    n_sub = chunk // SUBBATCH
    n_groups = chunk // LANES

    mesh = plsc.VectorSubcoreMesh(core_axis_name="c", subcore_axis_name="s",
                                  num_cores=NC, num_subcores=NS)

    @functools.partial(
        pl.kernel,
        out_type=jax.ShapeDtypeStruct((e_total,), jnp.float32),
        mesh=mesh,
        interpret=interpret,
        compiler_params=pltpu.CompilerParams(
            use_tc_tiling_on_sc=False, needs_layout_passes=False),
        scratch_types=[
            pltpu.VMEM((chunk,), jnp.int32),
            pltpu.VMEM((chunk,), jnp.int32),
            pltpu.VMEM((chunk, d), jnp.float32),
            pltpu.VMEM((chunk, d), jnp.float32),
            pltpu.VMEM((chunk,), jnp.float32),
            pltpu.SemaphoreType.DMA,
        ],
    )
    def k(z_hbm, src_hbm, dst_hbm, out_hbm, src_idx, dst_idx, src_rows,
          dst_rows, dots, sem):
        wid = lax.axis_index("s") * NC + lax.axis_index("c")
        wbase = wid * e_per_w
        lanes = lax.iota(jnp.int32, LANES)

        def do_chunk(g, _):
            base = wbase + g * chunk
            pltpu.sync_copy(src_hbm.at[pl.ds(base, chunk)], src_idx)
            pltpu.sync_copy(dst_hbm.at[pl.ds(base, chunk)], dst_idx)
            copies = []
            for j in range(n_sub):
                sl = pl.ds(j * SUBBATCH, SUBBATCH)
                copies.append(pltpu.async_copy(
                    z_hbm.at[src_idx.at[sl]], src_rows.at[sl], sem))
                copies.append(pltpu.async_copy(
                    z_hbm.at[dst_idx.at[sl]], dst_rows.at[sl], sem))
            for cp in copies:
                cp.wait()

            @plsc.parallel_loop(0, n_groups)
            def _group(g16):
                e0 = g16 * LANES
                out_vec = jnp.zeros((LANES,), jnp.float32)
                for k in range(LANES):
                    e = e0 + k
                    acc = (src_rows[e, pl.ds(0, LANES)]
                           * dst_rows[e, pl.ds(0, LANES)])
                    for c in range(1, d // LANES):
                        sl = pl.ds(c * LANES, LANES)
                        acc = acc + src_rows[e, sl] * dst_rows[e, sl]
                    s = jnp.sum(acc)
                    out_vec = jnp.where(lanes == k, s, out_vec)
                dots[pl.ds(e0, LANES)] = out_vec
            pltpu.sync_copy(dots, out_hbm.at[pl.ds(base, chunk)])
            return 0

        lax.fori_loop(0, n_chunks, do_chunk, 0)

    return k(z, srcs, dsts)


def _tc_loss(dots, e_pos, *, interpret=False):
    (e_total,) = dots.shape
    assert e_total == 2 * e_pos and e_pos % 128 == 0
    rows = e_pos // 128
    d3 = dots.reshape(2, rows, 128)
    inv = 1.0 / e_pos

    def body(d_ref, out_ref):
        p_pos = jax.nn.sigmoid(d_ref[0])
        p_neg = jax.nn.sigmoid(d_ref[1])
        q = jnp.maximum(1.0 - p_neg, 0.0)
        t = -jnp.log(p_pos + EPS) - jnp.log(q + EPS)
        out_ref[0, 0] = jnp.sum(t) * inv

    out = pl.pallas_call(
        body,
        out_specs=pl.BlockSpec(memory_space=pltpu.SMEM),
        out_shape=jax.ShapeDtypeStruct((1, 1), jnp.float32),
        interpret=interpret,
    )(d3)
    return out[0, 0]


def kernel(z, pos_edge_index, neg_edge_index, *, interpret=False):
    e_pos = pos_edge_index.shape[1]
    srcs = jnp.concatenate([pos_edge_index[0], neg_edge_index[0]])
    dsts = jnp.concatenate([pos_edge_index[1], neg_edge_index[1]])
    e_total = srcs.shape[0]
    e_per_w = e_total // NW
    chunk = 400 if e_per_w % 400 == 0 else e_per_w
    dots = _sc_dots(z, srcs, dsts, chunk=chunk, interpret=interpret)
    return _tc_loss(dots, e_pos, interpret=interpret)


# speedup vs baseline: 4.1427x; 1.2310x over previous
import functools

import jax
import jax.numpy as jnp
from jax import lax
from jax.experimental import pallas as pl
from jax.experimental.pallas import tpu as pltpu
from jax.experimental.pallas import tpu_sc as plsc

EPS = 1e-15
NC = 2
NS = 16
NW = NC * NS
LANES = 16


def _sc_dots(z, srcs, dsts, *, chunk, interpret=False):
    n, d = z.shape
    (e_total,) = srcs.shape
    assert e_total % NW == 0
    e_per_w = e_total // NW
    assert e_per_w % (2 * chunk) == 0 and chunk % LANES == 0 and chunk % 8 == 0
    assert chunk <= 128
    n_pairs = e_per_w // (2 * chunk)
    n_groups = chunk // LANES

    mesh = plsc.VectorSubcoreMesh(core_axis_name="c", subcore_axis_name="s",
                                  num_cores=NC, num_subcores=NS)

    @functools.partial(
        pl.kernel,
        out_type=jax.ShapeDtypeStruct((e_total,), jnp.float32),
        mesh=mesh,
        interpret=interpret,
        compiler_params=pltpu.CompilerParams(
            use_tc_tiling_on_sc=False, needs_layout_passes=False),
        scratch_types=[
            pltpu.VMEM((e_per_w,), jnp.int32),
            pltpu.VMEM((e_per_w,), jnp.int32),
            pltpu.VMEM((2, chunk, d), jnp.float32),
            pltpu.VMEM((2, chunk, d), jnp.float32),
            pltpu.VMEM((2, chunk), jnp.float32),
            pltpu.SemaphoreType.DMA((2,)),
            pltpu.SemaphoreType.DMA((2,)),
        ],
    )
    def k(z_hbm, src_hbm, dst_hbm, out_hbm, src_idx, dst_idx, src_rows,
          dst_rows, dots, gsem, ssem):
        wid = lax.axis_index("s") * NC + lax.axis_index("c")
        wbase = wid * e_per_w
        lanes = lax.iota(jnp.int32, LANES)

        pltpu.sync_copy(src_hbm.at[pl.ds(wbase, e_per_w)], src_idx)
        pltpu.sync_copy(dst_hbm.at[pl.ds(wbase, e_per_w)], dst_idx)

        def issue(g, slot):
            sl = pl.ds(g * chunk, chunk)
            pltpu.async_copy(z_hbm.at[src_idx.at[sl]], src_rows.at[slot],
                             gsem.at[slot])
            pltpu.async_copy(z_hbm.at[dst_idx.at[sl]], dst_rows.at[slot],
                             gsem.at[slot])

        def wait_gathers(slot):
            pltpu.make_async_copy(z_hbm.at[pl.ds(0, chunk)],
                                  src_rows.at[slot], gsem.at[slot]).wait()
            pltpu.make_async_copy(z_hbm.at[pl.ds(0, chunk)],
                                  dst_rows.at[slot], gsem.at[slot]).wait()

        def drain_scatter(slot):
            pltpu.make_async_copy(out_hbm.at[pl.ds(0, chunk)],
                                  dots.at[slot], ssem.at[slot]).wait()

        def compute(g, slot):
            sref = src_rows.at[slot]
            dref = dst_rows.at[slot]

            @plsc.parallel_loop(0, n_groups)
            def _group(g16):
                e0 = g16 * LANES
                out_vec = jnp.zeros((LANES,), jnp.float32)
                for kk in range(LANES):
                    e = e0 + kk
                    acc = sref[e, pl.ds(0, LANES)] * dref[e, pl.ds(0, LANES)]
                    for c in range(1, d // LANES):
                        sl = pl.ds(c * LANES, LANES)
                        acc = acc + sref[e, sl] * dref[e, sl]
                    s = jnp.sum(acc)
                    out_vec = jnp.where(lanes == kk, s, out_vec)
                dots[slot, pl.ds(e0, LANES)] = out_vec

            pltpu.async_copy(dots.at[slot],
                             out_hbm.at[pl.ds(wbase + g * chunk, chunk)],
                             ssem.at[slot])

        issue(0, 0)
        issue(1, 1)

        def pair(i, _):
            a = 2 * i
            wait_gathers(0)

            @pl.when(i > 0)
            def _():
                drain_scatter(0)

            compute(a, 0)

            @pl.when(i < n_pairs - 1)
            def _():
                issue(a + 2, 0)

            wait_gathers(1)

            @pl.when(i > 0)
            def _():
                drain_scatter(1)

            compute(a + 1, 1)

            @pl.when(i < n_pairs - 1)
            def _():
                issue(a + 3, 1)

            return 0

        lax.fori_loop(0, n_pairs, pair, 0)
        drain_scatter(0)
        drain_scatter(1)

    return k(z, srcs, dsts)


def _tc_loss(dots, e_pos, *, interpret=False):
    (e_total,) = dots.shape
    assert e_total == 2 * e_pos and e_pos % 128 == 0
    rows = e_pos // 128
    d3 = dots.reshape(2, rows, 128)
    inv = 1.0 / e_pos

    def body(d_ref, out_ref):
        p_pos = jax.nn.sigmoid(d_ref[0])
        p_neg = jax.nn.sigmoid(d_ref[1])
        q = jnp.maximum(1.0 - p_neg, 0.0)
        t = -jnp.log(p_pos + EPS) - jnp.log(q + EPS)
        out_ref[0, 0] = jnp.sum(t) * inv

    out = pl.pallas_call(
        body,
        out_specs=pl.BlockSpec(memory_space=pltpu.SMEM),
        out_shape=jax.ShapeDtypeStruct((1, 1), jnp.float32),
        interpret=interpret,
    )(d3)
    return out[0, 0]


def kernel(z, pos_edge_index, neg_edge_index, *, interpret=False):
    e_pos = pos_edge_index.shape[1]
    srcs = jnp.concatenate([pos_edge_index[0], neg_edge_index[0]])
    dsts = jnp.concatenate([pos_edge_index[1], neg_edge_index[1]])
    e_total = srcs.shape[0]
    e_per_w = e_total // NW
    chunk = 80 if e_per_w % 160 == 0 else e_per_w
    dots = _sc_dots(z, srcs, dsts, chunk=chunk, interpret=interpret)
    return _tc_loss(dots, e_pos, interpret=interpret)
